# baseline (speedup 1.0000x reference)
import functools

import jax
import jax.numpy as jnp
from jax import lax
from jax.experimental import pallas as pl
from jax.experimental.pallas import tpu as pltpu
from jax.experimental.pallas import tpu_sc as plsc

NC, NS = 2, 16
NW = NC * NS
CH = 128


def _proj_body(nf_ref, wa_ref, wb_ref, pa_ref, pb_ref):
    nf = nf_ref[...]
    pa_ref[...] = jnp.dot(nf, wa_ref[...], preferred_element_type=jnp.float32)
    pb_ref[...] = jnp.dot(nf, wb_ref[...], preferred_element_type=jnp.float32)


def _ln_rows(h, g, b, eps=1e-5):
    mu = jnp.mean(h, axis=-1, keepdims=True)
    var = jnp.mean((h - mu) ** 2, axis=-1, keepdims=True)
    return (h - mu) / jnp.sqrt(var + eps) * g + b


def _edge_body(g1_ref, g2_ref, ef_ref, wc_ref, be1_ref, we2_ref, be2_ref,
               ge_ref, bne_ref, *refs):
    ne_ref, eo_ref = refs[-2], refs[-1]
    ef = ef_ref[...]
    h = (g1_ref[...] + g2_ref[...] + be1_ref[...]
         + jnp.dot(ef, wc_ref[...], preferred_element_type=jnp.float32))
    h = jnp.maximum(h, 0.0)
    h2 = jnp.dot(h, we2_ref[...], preferred_element_type=jnp.float32) + be2_ref[...]
    ne = _ln_rows(h2, ge_ref[...], bne_ref[...])
    ne_ref[...] = ne
    eo_ref[...] = ne + ef


def _node_body(nf_ref, agga_ref, aggb_ref, wx_ref, wg_ref, bn1_ref, wn2_ref,
               bn2_ref, gn_ref, bnn_ref, out_ref):
    nf = nf_ref[...]
    agg = agga_ref[0] + agga_ref[1] + aggb_ref[0] + aggb_ref[1]
    h = (jnp.dot(nf, wx_ref[...], preferred_element_type=jnp.float32)
         + jnp.dot(agg, wg_ref[...], preferred_element_type=jnp.float32)
         + bn1_ref[...])
    h = jnp.maximum(h, 0.0)
    h2 = jnp.dot(h, wn2_ref[...], preferred_element_type=jnp.float32) + bn2_ref[...]
    out_ref[...] = _ln_rows(h2, gn_ref[...], bnn_ref[...]) + nf


NSLOT = 3


def _make_gather(E, D):
    per = E // NW
    nch = per // CH
    tail = per - nch * CH
    ntrip = nch // NSLOT
    mesh = plsc.VectorSubcoreMesh(core_axis_name="c", subcore_axis_name="s",
                                  num_cores=NC, num_subcores=NS)

    scratch = []
    for _ in range(NSLOT):
        scratch += [pltpu.VMEM((CH,), jnp.int32),
                    pltpu.VMEM((CH, D), jnp.float32),
                    pltpu.VMEM((CH,), jnp.int32),
                    pltpu.VMEM((CH, D), jnp.float32),
                    pltpu.SemaphoreType.DMA,
                    pltpu.SemaphoreType.DMA,
                    pltpu.SemaphoreType.DMA,
                    pltpu.SemaphoreType.DMA]

    @functools.partial(
        pl.kernel,
        out_type=(jax.ShapeDtypeStruct((E, D), jnp.float32),
                  jax.ShapeDtypeStruct((E, D), jnp.float32)),
        mesh=mesh,
        scratch_types=scratch,
    )
    def gather_k(pa_hbm, pb_hbm, src_hbm, dst_hbm, dep_hbm, g1_hbm, g2_hbm,
                 *bufs):
        slot = [bufs[8 * m:8 * (m + 1)] for m in range(NSLOT)]
        wid = lax.axis_index("s") * NC + lax.axis_index("c")
        base0 = wid * per

        def issue(c, m):
            idx1, rows1, idx2, rows2, gs1, gs2, _, _ = slot[m]
            base = base0 + c * CH
            pltpu.sync_copy(src_hbm.at[pl.ds(base, CH)], idx1)
            pltpu.sync_copy(dst_hbm.at[pl.ds(base, CH)], idx2)
            pltpu.async_copy(pa_hbm.at[idx1], rows1, gs1)
            pltpu.async_copy(pb_hbm.at[idx2], rows2, gs2)

        def wait_gather(m):
            idx1, rows1, idx2, rows2, gs1, gs2, _, _ = slot[m]
            pltpu.make_async_copy(pa_hbm.at[idx1], rows1, gs1).wait()
            pltpu.make_async_copy(pb_hbm.at[idx2], rows2, gs2).wait()

        def writeback(c, m):
            _, rows1, _, rows2, _, _, ws1, ws2 = slot[m]
            base = base0 + c * CH
            pltpu.async_copy(rows1, g1_hbm.at[pl.ds(base, CH)], ws1)
            pltpu.async_copy(rows2, g2_hbm.at[pl.ds(base, CH)], ws2)

        def wait_writeback(c, m):
            _, rows1, _, rows2, _, _, ws1, ws2 = slot[m]
            base = base0 + c * CH
            pltpu.make_async_copy(rows1, g1_hbm.at[pl.ds(base, CH)], ws1).wait()
            pltpu.make_async_copy(rows2, g2_hbm.at[pl.ds(base, CH)], ws2).wait()

        for m in range(NSLOT):
            issue(m, m)

        def body(j, _):
            c0 = j * NSLOT
            for m in range(NSLOT):
                wait_gather(m)
                writeback(c0 + m, m)

            @pl.when(j < ntrip - 1)
            def _():
                for m in range(NSLOT):
                    wait_writeback(c0 + m, m)
                    issue(c0 + NSLOT + m, m)

            return 0

        lax.fori_loop(0, ntrip, body, 0)
        for m in range(NSLOT):
            wait_writeback((ntrip - 1) * NSLOT + m, m)

        idx1, rows1, idx2, rows2, gs1, gs2, _, _ = slot[0]
        base = base0 + nch * CH
        ti1, ti2 = idx1.at[pl.ds(0, tail)], idx2.at[pl.ds(0, tail)]
        tr1, tr2 = rows1.at[pl.ds(0, tail)], rows2.at[pl.ds(0, tail)]
        pltpu.sync_copy(src_hbm.at[pl.ds(base, tail)], ti1)
        pltpu.sync_copy(dst_hbm.at[pl.ds(base, tail)], ti2)
        pltpu.async_copy(pa_hbm.at[ti1], tr1, gs1).wait()
        pltpu.async_copy(pb_hbm.at[ti2], tr2, gs2).wait()
        pltpu.sync_copy(tr1, g1_hbm.at[pl.ds(base, tail)])
        pltpu.sync_copy(tr2, g2_hbm.at[pl.ds(base, tail)])

    return gather_k


def _make_scatter(E, N, D):
    per = E // NW
    nch = per // CH
    tail = per - nch * CH
    rows_per_tile = N // NS
    SR = 125
    nsc = rows_per_tile // SR
    mesh = plsc.VectorSubcoreMesh(core_axis_name="c", subcore_axis_name="s",
                                  num_cores=NC, num_subcores=NS)

    @functools.partial(
        pl.kernel,
        out_type=jax.ShapeDtypeStruct((NC, NS, nsc, SR, D), jnp.float32),
        mesh=mesh,
        scratch_types=[
            pltpu.VMEM((CH,), jnp.int32),
            pltpu.VMEM((CH, D), jnp.float32),
            pltpu.VMEM((CH,), jnp.int32),
            pltpu.VMEM((CH, D), jnp.float32),
            pltpu.VMEM((SR, D), jnp.float32),
            pltpu.VMEM_SHARED((N, D), jnp.float32),
            pltpu.VMEM((tail or 8,), jnp.int32),
            pltpu.SemaphoreType.DMA,
            pltpu.SemaphoreType.DMA,
            pltpu.SemaphoreType.DMA,
            pltpu.SemaphoreType.DMA,
        ],
    )
    def scatter_k(ne_hbm, dst_hbm, dep_hbm, out_hbm, idx0, rows0, idx1, rows1,
                  stage, agg_sh, tidx, ls0, ls1, ss0, ss1):
        cid = lax.axis_index("c")
        sid = lax.axis_index("s")
        wid = sid * NC + cid
        base0 = wid * per
        row0 = sid * rows_per_tile
        idx = [idx0, idx1]
        rows = [rows0, rows1]
        lsem = [ls0, ls1]
        ssem = [ss0, ss1]

        def zbody(i, _):
            r = i // (D // 16)
            c = (i % (D // 16)) * 16
            stage[r, pl.ds(c, 16)] = jnp.zeros((16,), jnp.float32)
            return 0

        lax.fori_loop(0, SR * (D // 16), zbody, 0)
        for k in range(nsc):
            pltpu.sync_copy(stage, agg_sh.at[pl.ds(row0 + k * SR, SR)])
        plsc.subcore_barrier()

        def load(c, m):
            base = base0 + c * CH
            pltpu.async_copy(dst_hbm.at[pl.ds(base, CH)], idx[m], lsem[m])
            pltpu.async_copy(ne_hbm.at[pl.ds(base, CH)], rows[m], lsem[m])

        def wait_load(c, m):
            base = base0 + c * CH
            pltpu.make_async_copy(dst_hbm.at[pl.ds(base, CH)], idx[m], lsem[m]).wait()
            pltpu.make_async_copy(ne_hbm.at[pl.ds(base, CH)], rows[m], lsem[m]).wait()

        def scat(m):
            pltpu.async_copy(rows[m], agg_sh.at[idx[m]], ssem[m], add=True)

        def wait_scat(m):
            pltpu.make_async_copy(rows[m], agg_sh.at[idx[m]], ssem[m]).wait()

        npair = nch // 2
        load(0, 0)
        load(1, 1)

        def body(j, _):
            c0 = 2 * j
            for m in range(2):
                wait_load(c0 + m, m)
                scat(m)
            for m in range(2):
                wait_scat(m)
                extra = nch % 2 if m == 0 else 0

                @pl.when(j < npair - 1 + extra)
                def _():
                    load(c0 + 2 + m, m)

            return 0

        lax.fori_loop(0, npair, body, 0)

        if nch % 2:
            wait_load(nch - 1, 0)
            scat(0)
            wait_scat(0)

        if tail:
            base = base0 + nch * CH
            tr = rows0.at[pl.ds(0, tail)]
            pltpu.sync_copy(dst_hbm.at[pl.ds(base, tail)], tidx)
            pltpu.sync_copy(ne_hbm.at[pl.ds(base, tail)], tr)
            pltpu.sync_copy(tr, agg_sh.at[tidx], add=True)
        plsc.subcore_barrier()

        for k in range(nsc):
            pltpu.sync_copy(agg_sh.at[pl.ds(row0 + k * SR, SR)], stage)
            pltpu.sync_copy(stage, out_hbm.at[cid, sid, k])

    return scatter_k


def kernel(node_features, edge_features, edge_index, We1, be1, We2, be2,
           ge, bne, Wn1, bn1, Wn2, bn2, gn, bnn):
    N, D = node_features.shape
    E, DE = edge_features.shape
    src = edge_index[0]
    dst = edge_index[1]

    Wa, Wb, Wc = We1[:D], We1[D:2 * D], We1[2 * D:]
    Wn1x, Wn1g = Wn1[:D], Wn1[D:]
    be1r, be2r = be1.reshape(1, DE), be2.reshape(1, DE)
    ger, bner = ge.reshape(1, DE), bne.reshape(1, DE)
    bn1r, bn2r = bn1.reshape(1, D), bn2.reshape(1, D)
    gnr, bnnr = gn.reshape(1, D), bnn.reshape(1, D)

    BN = 1000
    full = lambda s: pl.BlockSpec(s, lambda i: tuple(0 for _ in s))

    pa, pb = pl.pallas_call(
        _proj_body,
        grid=(N // BN,),
        in_specs=[
            pl.BlockSpec((BN, D), lambda i: (i, 0)),
            full((D, DE)), full((D, DE)),
        ],
        out_specs=[pl.BlockSpec((BN, DE), lambda i: (i, 0)),
                   pl.BlockSpec((BN, DE), lambda i: (i, 0))],
        out_shape=[jax.ShapeDtypeStruct((N, DE), jnp.float32),
                   jax.ShapeDtypeStruct((N, DE), jnp.float32)],
    )(node_features, Wa, Wb)

    E2 = E // 2
    BE = 3200
    nb = E2 // BE
    gather = _make_gather(E2, DE)
    scatter = _make_scatter(E2, N, DE)

    def edge_call(phase, g1, g2, ef, eo_prev):
        specs = [
            pl.BlockSpec((BE, DE), lambda i: (i, 0)),
            pl.BlockSpec((BE, DE), lambda i: (i, 0)),
            pl.BlockSpec((BE, DE), lambda i: (i, 0)),
            full((DE, DE)), full((1, DE)), full((DE, DE)), full((1, DE)),
            full((1, DE)), full((1, DE)),
        ]
        args = [g1, g2, ef, Wc, be1r, We2, be2r, ger, bner]
        aliases = {}
        if eo_prev is not None:
            specs.append(pl.BlockSpec(memory_space=pl.ANY))
            args.append(eo_prev)
            aliases = {9: 1}
        off = phase * nb
        return pl.pallas_call(
            _edge_body,
            grid=(nb,),
            in_specs=specs,
            out_specs=[pl.BlockSpec((BE, DE), lambda i: (i, 0)),
                       pl.BlockSpec((BE, DE), lambda i, o=off: (i + o, 0))],
            out_shape=[jax.ShapeDtypeStruct((E2, DE), jnp.float32),
                       jax.ShapeDtypeStruct((E, DE), jnp.float32)],
            input_output_aliases=aliases,
        )(*args)

    g1a, g2a = gather(pa, pb, src[:E2], dst[:E2], pa[:1, :8])
    g1b, g2b = gather(pa, pb, src[E2:], dst[E2:], g1a[:1, :8])
    ne_a, eo_a = edge_call(0, g1a, g2a, edge_features[:E2], None)
    parts_a = scatter(ne_a, dst[:E2], g1b[:1, :8]).reshape(NC, N, DE)
    ne_b, edge_out = edge_call(1, g1b, g2b, edge_features[E2:], eo_a)
    parts_b = scatter(ne_b, dst[E2:], parts_a[:1, :1, :8].reshape(1, 8)
                      ).reshape(NC, N, DE)

    node_out = pl.pallas_call(
        _node_body,
        grid=(N // BN,),
        in_specs=[
            pl.BlockSpec((BN, D), lambda i: (i, 0)),
            pl.BlockSpec((NC, BN, DE), lambda i: (0, i, 0)),
            pl.BlockSpec((NC, BN, DE), lambda i: (0, i, 0)),
            full((D, D)), full((DE, D)), full((1, D)),
            full((D, D)), full((1, D)), full((1, D)), full((1, D)),
        ],
        out_specs=pl.BlockSpec((BN, D), lambda i: (i, 0)),
        out_shape=jax.ShapeDtypeStruct((N, D), jnp.float32),
    )(node_features, parts_a, parts_b, Wn1x, Wn1g, bn1r, Wn2, bn2r, gnr, bnnr)

    return node_out, edge_out


# speedup vs baseline: 1.1871x; 1.1871x over previous
import functools

import jax
import jax.numpy as jnp
from jax import lax
from jax.experimental import pallas as pl
from jax.experimental.pallas import tpu as pltpu
from jax.experimental.pallas import tpu_sc as plsc

NC, NS = 2, 16
NW = NC * NS
CH = 128


def _proj_body(nf_ref, wa_ref, wb_ref, pa_ref, pb_ref):
    nf = nf_ref[...]
    pa_ref[...] = jnp.dot(nf, wa_ref[...], preferred_element_type=jnp.float32)
    pb_ref[...] = jnp.dot(nf, wb_ref[...], preferred_element_type=jnp.float32)


def _ln_rows(h, g, b, eps=1e-5):
    mu = jnp.mean(h, axis=-1, keepdims=True)
    var = jnp.mean((h - mu) ** 2, axis=-1, keepdims=True)
    return (h - mu) / jnp.sqrt(var + eps) * g + b


def _edge_body(g_ref, ef_ref, wc_ref, be1_ref, we2_ref, be2_ref,
               ge_ref, bne_ref, ne_ref, eo_ref):
    ef = ef_ref[...]
    h = (g_ref[...] + be1_ref[...]
         + jnp.dot(ef, wc_ref[...], preferred_element_type=jnp.float32))
    h = jnp.maximum(h, 0.0)
    h2 = jnp.dot(h, we2_ref[...], preferred_element_type=jnp.float32) + be2_ref[...]
    ne = _ln_rows(h2, ge_ref[...], bne_ref[...])
    ne_ref[...] = ne
    eo_ref[...] = ne + ef


def _node_body(nf_ref, agg_ref, wx_ref, wg_ref, bn1_ref, wn2_ref,
               bn2_ref, gn_ref, bnn_ref, out_ref):
    nf = nf_ref[...]
    agg = agg_ref[0] + agg_ref[1]
    h = (jnp.dot(nf, wx_ref[...], preferred_element_type=jnp.float32)
         + jnp.dot(agg, wg_ref[...], preferred_element_type=jnp.float32)
         + bn1_ref[...])
    h = jnp.maximum(h, 0.0)
    h2 = jnp.dot(h, wn2_ref[...], preferred_element_type=jnp.float32) + bn2_ref[...]
    out_ref[...] = _ln_rows(h2, gn_ref[...], bnn_ref[...]) + nf


NSLOT = 3


def _make_gather(E, D):
    per = E // NW
    nch = per // CH
    tail = per - nch * CH
    ntrip = nch // NSLOT
    mesh = plsc.VectorSubcoreMesh(core_axis_name="c", subcore_axis_name="s",
                                  num_cores=NC, num_subcores=NS)

    scratch = []
    for _ in range(NSLOT):
        scratch += [pltpu.VMEM((CH,), jnp.int32),
                    pltpu.VMEM((CH, D), jnp.float32),
                    pltpu.VMEM((CH,), jnp.int32),
                    pltpu.VMEM((CH, D), jnp.float32),
                    pltpu.SemaphoreType.DMA,
                    pltpu.SemaphoreType.DMA,
                    pltpu.SemaphoreType.DMA]

    @functools.partial(
        pl.kernel,
        out_type=jax.ShapeDtypeStruct((E, D), jnp.float32),
        mesh=mesh,
        scratch_types=scratch,
    )
    def gather_k(pa_hbm, pb_hbm, src_hbm, dst_hbm, g_hbm, *bufs):
        slot = [bufs[7 * m:7 * (m + 1)] for m in range(NSLOT)]
        wid = lax.axis_index("s") * NC + lax.axis_index("c")
        base0 = wid * per

        def issue(c, m):
            idx1, rows1, idx2, rows2, gs1, gs2, _ = slot[m]
            base = base0 + c * CH
            pltpu.sync_copy(src_hbm.at[pl.ds(base, CH)], idx1)
            pltpu.sync_copy(dst_hbm.at[pl.ds(base, CH)], idx2)
            pltpu.async_copy(pa_hbm.at[idx1], rows1, gs1)
            pltpu.async_copy(pb_hbm.at[idx2], rows2, gs2)

        def wait_gather(m):
            idx1, rows1, idx2, rows2, gs1, gs2, _ = slot[m]
            pltpu.make_async_copy(pa_hbm.at[idx1], rows1, gs1).wait()
            pltpu.make_async_copy(pb_hbm.at[idx2], rows2, gs2).wait()

        def fuse_add(m):
            _, rows1, _, rows2, _, _, _ = slot[m]

            def rbody(r, _):
                for c in range(D // 16):
                    sl = pl.ds(16 * c, 16)
                    plsc.addupdate(rows1.at[r, sl], rows2[r, sl])
                return 0

            lax.fori_loop(0, CH, rbody, 0)

        def writeback(c, m):
            _, rows1, _, _, _, _, ws = slot[m]
            base = base0 + c * CH
            pltpu.async_copy(rows1, g_hbm.at[pl.ds(base, CH)], ws)

        def wait_writeback(c, m):
            _, rows1, _, _, _, _, ws = slot[m]
            base = base0 + c * CH
            pltpu.make_async_copy(rows1, g_hbm.at[pl.ds(base, CH)], ws).wait()

        for m in range(NSLOT):
            issue(m, m)

        def body(j, _):
            c0 = j * NSLOT
            for m in range(NSLOT):
                wait_gather(m)
                fuse_add(m)
                writeback(c0 + m, m)

            @pl.when(j < ntrip - 1)
            def _():
                for m in range(NSLOT):
                    wait_writeback(c0 + m, m)
                    issue(c0 + NSLOT + m, m)

            return 0

        lax.fori_loop(0, ntrip, body, 0)
        for m in range(NSLOT):
            wait_writeback((ntrip - 1) * NSLOT + m, m)

        idx1, rows1, idx2, rows2, gs1, gs2, _ = slot[0]
        base = base0 + nch * CH
        ti1, ti2 = idx1.at[pl.ds(0, tail)], idx2.at[pl.ds(0, tail)]
        tr1, tr2 = rows1.at[pl.ds(0, tail)], rows2.at[pl.ds(0, tail)]
        pltpu.sync_copy(src_hbm.at[pl.ds(base, tail)], ti1)
        pltpu.sync_copy(dst_hbm.at[pl.ds(base, tail)], ti2)
        pltpu.async_copy(pa_hbm.at[ti1], tr1, gs1).wait()
        pltpu.async_copy(pb_hbm.at[ti2], tr2, gs2).wait()

        def tbody(r, _):
            for c in range(D // 16):
                sl = pl.ds(16 * c, 16)
                plsc.addupdate(rows1.at[r, sl], rows2[r, sl])
            return 0

        lax.fori_loop(0, tail, tbody, 0)
        pltpu.sync_copy(tr1, g_hbm.at[pl.ds(base, tail)])

    return gather_k


def _make_scatter(E, N, D):
    per = E // NW
    nch = per // CH
    tail = per - nch * CH
    rows_per_tile = N // NS
    SR = 125
    nsc = rows_per_tile // SR
    mesh = plsc.VectorSubcoreMesh(core_axis_name="c", subcore_axis_name="s",
                                  num_cores=NC, num_subcores=NS)

    @functools.partial(
        pl.kernel,
        out_type=jax.ShapeDtypeStruct((NC, NS, nsc, SR, D), jnp.float32),
        mesh=mesh,
        scratch_types=[
            pltpu.VMEM((CH,), jnp.int32),
            pltpu.VMEM((CH, D), jnp.float32),
            pltpu.VMEM((CH,), jnp.int32),
            pltpu.VMEM((CH, D), jnp.float32),
            pltpu.VMEM((SR, D), jnp.float32),
            pltpu.VMEM_SHARED((N, D), jnp.float32),
            pltpu.VMEM((tail or 8,), jnp.int32),
            pltpu.SemaphoreType.DMA,
            pltpu.SemaphoreType.DMA,
            pltpu.SemaphoreType.DMA,
            pltpu.SemaphoreType.DMA,
        ],
    )
    def scatter_k(ne_hbm, dst_hbm, out_hbm, idx0, rows0, idx1, rows1,
                  stage, agg_sh, tidx, ls0, ls1, ss0, ss1):
        cid = lax.axis_index("c")
        sid = lax.axis_index("s")
        wid = sid * NC + cid
        base0 = wid * per
        row0 = sid * rows_per_tile
        idx = [idx0, idx1]
        rows = [rows0, rows1]
        lsem = [ls0, ls1]
        ssem = [ss0, ss1]

        def zbody(i, _):
            r = i // (D // 16)
            c = (i % (D // 16)) * 16
            stage[r, pl.ds(c, 16)] = jnp.zeros((16,), jnp.float32)
            return 0

        lax.fori_loop(0, SR * (D // 16), zbody, 0)
        for k in range(nsc):
            pltpu.sync_copy(stage, agg_sh.at[pl.ds(row0 + k * SR, SR)])
        plsc.subcore_barrier()

        def load(c, m):
            base = base0 + c * CH
            pltpu.async_copy(dst_hbm.at[pl.ds(base, CH)], idx[m], lsem[m])
            pltpu.async_copy(ne_hbm.at[pl.ds(base, CH)], rows[m], lsem[m])

        def wait_load(c, m):
            base = base0 + c * CH
            pltpu.make_async_copy(dst_hbm.at[pl.ds(base, CH)], idx[m], lsem[m]).wait()
            pltpu.make_async_copy(ne_hbm.at[pl.ds(base, CH)], rows[m], lsem[m]).wait()

        def scat(m):
            pltpu.async_copy(rows[m], agg_sh.at[idx[m]], ssem[m], add=True)

        def wait_scat(m):
            pltpu.make_async_copy(rows[m], agg_sh.at[idx[m]], ssem[m]).wait()

        npair = nch // 2
        load(0, 0)
        load(1, 1)

        def body(j, _):
            c0 = 2 * j
            for m in range(2):
                wait_load(c0 + m, m)
                scat(m)
            for m in range(2):
                wait_scat(m)
                extra = nch % 2 if m == 0 else 0

                @pl.when(j < npair - 1 + extra)
                def _():
                    load(c0 + 2 + m, m)

            return 0

        lax.fori_loop(0, npair, body, 0)

        if nch % 2:
            wait_load(nch - 1, 0)
            scat(0)
            wait_scat(0)

        if tail:
            base = base0 + nch * CH
            tr = rows0.at[pl.ds(0, tail)]
            pltpu.sync_copy(dst_hbm.at[pl.ds(base, tail)], tidx)
            pltpu.sync_copy(ne_hbm.at[pl.ds(base, tail)], tr)
            pltpu.sync_copy(tr, agg_sh.at[tidx], add=True)
        plsc.subcore_barrier()

        for k in range(nsc):
            pltpu.sync_copy(agg_sh.at[pl.ds(row0 + k * SR, SR)], stage)
            pltpu.sync_copy(stage, out_hbm.at[cid, sid, k])

    return scatter_k


def kernel(node_features, edge_features, edge_index, We1, be1, We2, be2,
           ge, bne, Wn1, bn1, Wn2, bn2, gn, bnn):
    N, D = node_features.shape
    E, DE = edge_features.shape
    src = edge_index[0]
    dst = edge_index[1]

    Wa, Wb, Wc = We1[:D], We1[D:2 * D], We1[2 * D:]
    Wn1x, Wn1g = Wn1[:D], Wn1[D:]
    be1r, be2r = be1.reshape(1, DE), be2.reshape(1, DE)
    ger, bner = ge.reshape(1, DE), bne.reshape(1, DE)
    bn1r, bn2r = bn1.reshape(1, D), bn2.reshape(1, D)
    gnr, bnnr = gn.reshape(1, D), bnn.reshape(1, D)

    BN = 1000
    full = lambda s: pl.BlockSpec(s, lambda i: tuple(0 for _ in s))

    pa, pb = pl.pallas_call(
        _proj_body,
        grid=(N // BN,),
        in_specs=[
            pl.BlockSpec((BN, D), lambda i: (i, 0)),
            full((D, DE)), full((D, DE)),
        ],
        out_specs=[pl.BlockSpec((BN, DE), lambda i: (i, 0)),
                   pl.BlockSpec((BN, DE), lambda i: (i, 0))],
        out_shape=[jax.ShapeDtypeStruct((N, DE), jnp.float32),
                   jax.ShapeDtypeStruct((N, DE), jnp.float32)],
    )(node_features, Wa, Wb)

    g = _make_gather(E, DE)(pa, pb, src, dst)

    BE = 3200
    new_edge, edge_out = pl.pallas_call(
        _edge_body,
        grid=(E // BE,),
        in_specs=[
            pl.BlockSpec((BE, DE), lambda i: (i, 0)),
            pl.BlockSpec((BE, DE), lambda i: (i, 0)),
            full((DE, DE)), full((1, DE)), full((DE, DE)), full((1, DE)),
            full((1, DE)), full((1, DE)),
        ],
        out_specs=[pl.BlockSpec((BE, DE), lambda i: (i, 0)),
                   pl.BlockSpec((BE, DE), lambda i: (i, 0))],
        out_shape=[jax.ShapeDtypeStruct((E, DE), jnp.float32),
                   jax.ShapeDtypeStruct((E, DE), jnp.float32)],
    )(g, edge_features, Wc, be1r, We2, be2r, ger, bner)

    parts = _make_scatter(E, N, DE)(new_edge, dst).reshape(NC, N, DE)

    node_out = pl.pallas_call(
        _node_body,
        grid=(N // BN,),
        in_specs=[
            pl.BlockSpec((BN, D), lambda i: (i, 0)),
            pl.BlockSpec((NC, BN, DE), lambda i: (0, i, 0)),
            full((D, D)), full((DE, D)), full((1, D)),
            full((D, D)), full((1, D)), full((1, D)), full((1, D)),
        ],
        out_specs=pl.BlockSpec((BN, D), lambda i: (i, 0)),
        out_shape=jax.ShapeDtypeStruct((N, D), jnp.float32),
    )(node_features, parts, Wn1x, Wn1g, bn1r, Wn2, bn2r, gnr, bnnr)

    return node_out, edge_out


# speedup vs baseline: 1.3198x; 1.1118x over previous
import functools

import jax
import jax.numpy as jnp
from jax import lax
from jax.experimental import pallas as pl
from jax.experimental.pallas import tpu as pltpu
from jax.experimental.pallas import tpu_sc as plsc

NC, NS = 2, 16
NW = NC * NS
CH = 128


def _proj_body(nf_ref, wa_ref, wb_ref, pa_ref, pb_ref):
    nf = nf_ref[...]
    pa_ref[...] = jnp.dot(nf, wa_ref[...], preferred_element_type=jnp.float32)
    pb_ref[...] = jnp.dot(nf, wb_ref[...], preferred_element_type=jnp.float32)


def _ln_rows(h, g, b, eps=1e-5):
    mu = jnp.mean(h, axis=-1, keepdims=True)
    var = jnp.mean((h - mu) ** 2, axis=-1, keepdims=True)
    return (h - mu) / jnp.sqrt(var + eps) * g + b


def _edge_body(g_ref, ef_ref, wc_ref, be1_ref, we2_ref, be2_ref,
               ge_ref, bne_ref, *refs):
    ne_ref, eo_ref = refs[-2], refs[-1]
    ef = ef_ref[...]
    h = (g_ref[...] + be1_ref[...]
         + jnp.dot(ef, wc_ref[...], preferred_element_type=jnp.float32))
    h = jnp.maximum(h, 0.0)
    h2 = jnp.dot(h, we2_ref[...], preferred_element_type=jnp.float32) + be2_ref[...]
    ne = _ln_rows(h2, ge_ref[...], bne_ref[...])
    ne_ref[...] = ne
    eo_ref[...] = ne + ef


def _node_body(nf_ref, agga_ref, aggb_ref, wx_ref, wg_ref, bn1_ref, wn2_ref,
               bn2_ref, gn_ref, bnn_ref, out_ref):
    nf = nf_ref[...]
    agg = agga_ref[0] + agga_ref[1] + aggb_ref[0] + aggb_ref[1]
    h = (jnp.dot(nf, wx_ref[...], preferred_element_type=jnp.float32)
         + jnp.dot(agg, wg_ref[...], preferred_element_type=jnp.float32)
         + bn1_ref[...])
    h = jnp.maximum(h, 0.0)
    h2 = jnp.dot(h, wn2_ref[...], preferred_element_type=jnp.float32) + bn2_ref[...]
    out_ref[...] = _ln_rows(h2, gn_ref[...], bnn_ref[...]) + nf


NSLOT = 3


def _make_gather(E, D, e0=0, Eout=None):
    per = E // NW
    nch = per // CH
    tail = per - nch * CH
    ntrip = nch // NSLOT
    Eout = E if Eout is None else Eout
    mesh = plsc.VectorSubcoreMesh(core_axis_name="c", subcore_axis_name="s",
                                  num_cores=NC, num_subcores=NS)

    scratch = []
    for _ in range(NSLOT):
        scratch += [pltpu.VMEM((CH,), jnp.int32),
                    pltpu.VMEM((CH, D), jnp.float32),
                    pltpu.VMEM((CH,), jnp.int32),
                    pltpu.VMEM((CH, D), jnp.float32),
                    pltpu.SemaphoreType.DMA,
                    pltpu.SemaphoreType.DMA,
                    pltpu.SemaphoreType.DMA]

    @functools.partial(
        pl.kernel,
        out_type=jax.ShapeDtypeStruct((Eout, D), jnp.float32),
        mesh=mesh,
        scratch_types=scratch,
    )
    def gather_k(pa_hbm, pb_hbm, src_hbm, dst_hbm, g_hbm, *bufs):
        slot = [bufs[7 * m:7 * (m + 1)] for m in range(NSLOT)]
        wid = lax.axis_index("s") * NC + lax.axis_index("c")
        base0 = wid * per
        gsrc0 = e0 + base0

        def issue(c, m):
            idx1, rows1, idx2, rows2, gs1, gs2, _ = slot[m]
            base = gsrc0 + c * CH
            pltpu.sync_copy(src_hbm.at[pl.ds(base, CH)], idx1)
            pltpu.sync_copy(dst_hbm.at[pl.ds(base, CH)], idx2)
            pltpu.async_copy(pa_hbm.at[idx1], rows1, gs1)
            pltpu.async_copy(pb_hbm.at[idx2], rows2, gs2)

        def wait_gather(m):
            idx1, rows1, idx2, rows2, gs1, gs2, _ = slot[m]
            pltpu.make_async_copy(pa_hbm.at[idx1], rows1, gs1).wait()
            pltpu.make_async_copy(pb_hbm.at[idx2], rows2, gs2).wait()

        def fuse_add(m):
            _, rows1, _, rows2, _, _, _ = slot[m]

            def rbody(r, _):
                for c in range(D // 16):
                    sl = pl.ds(16 * c, 16)
                    plsc.addupdate(rows1.at[r, sl], rows2[r, sl])
                return 0

            lax.fori_loop(0, CH, rbody, 0)

        def writeback(c, m):
            _, rows1, _, _, _, _, ws = slot[m]
            base = base0 + c * CH
            pltpu.async_copy(rows1, g_hbm.at[pl.ds(base, CH)], ws)

        def wait_writeback(c, m):
            _, rows1, _, _, _, _, ws = slot[m]
            base = base0 + c * CH
            pltpu.make_async_copy(rows1, g_hbm.at[pl.ds(base, CH)], ws).wait()

        for m in range(NSLOT):
            issue(m, m)

        def body(j, _):
            c0 = j * NSLOT
            for m in range(NSLOT):
                wait_gather(m)
                fuse_add(m)
                writeback(c0 + m, m)

            @pl.when(j < ntrip - 1)
            def _():
                for m in range(NSLOT):
                    wait_writeback(c0 + m, m)
                    issue(c0 + NSLOT + m, m)

            return 0

        lax.fori_loop(0, ntrip, body, 0)
        for m in range(NSLOT):
            wait_writeback((ntrip - 1) * NSLOT + m, m)

        idx1, rows1, idx2, rows2, gs1, gs2, _ = slot[0]
        base = base0 + nch * CH
        gsbase = gsrc0 + nch * CH
        ti1, ti2 = idx1.at[pl.ds(0, tail)], idx2.at[pl.ds(0, tail)]
        tr1, tr2 = rows1.at[pl.ds(0, tail)], rows2.at[pl.ds(0, tail)]
        pltpu.sync_copy(src_hbm.at[pl.ds(gsbase, tail)], ti1)
        pltpu.sync_copy(dst_hbm.at[pl.ds(gsbase, tail)], ti2)
        pltpu.async_copy(pa_hbm.at[ti1], tr1, gs1).wait()
        pltpu.async_copy(pb_hbm.at[ti2], tr2, gs2).wait()

        def tbody(r, _):
            for c in range(D // 16):
                sl = pl.ds(16 * c, 16)
                plsc.addupdate(rows1.at[r, sl], rows2[r, sl])
            return 0

        lax.fori_loop(0, tail, tbody, 0)
        pltpu.sync_copy(tr1, g_hbm.at[pl.ds(base, tail)])

    return gather_k


def _make_scatter(E, N, D, e0=0):
    per = E // NW
    nch = per // CH
    tail = per - nch * CH
    rows_per_tile = N // NS
    SR = 125
    nsc = rows_per_tile // SR
    mesh = plsc.VectorSubcoreMesh(core_axis_name="c", subcore_axis_name="s",
                                  num_cores=NC, num_subcores=NS)

    @functools.partial(
        pl.kernel,
        out_type=jax.ShapeDtypeStruct((NC, NS, nsc, SR, D), jnp.float32),
        mesh=mesh,
        scratch_types=[
            pltpu.VMEM((CH,), jnp.int32),
            pltpu.VMEM((CH, D), jnp.float32),
            pltpu.VMEM((CH,), jnp.int32),
            pltpu.VMEM((CH, D), jnp.float32),
            pltpu.VMEM((SR, D), jnp.float32),
            pltpu.VMEM_SHARED((N, D), jnp.float32),
            pltpu.VMEM((tail or 8,), jnp.int32),
            pltpu.SemaphoreType.DMA,
            pltpu.SemaphoreType.DMA,
            pltpu.SemaphoreType.DMA,
            pltpu.SemaphoreType.DMA,
        ],
    )
    def scatter_k(ne_hbm, dst_hbm, out_hbm, idx0, rows0, idx1, rows1,
                  stage, agg_sh, tidx, ls0, ls1, ss0, ss1):
        cid = lax.axis_index("c")
        sid = lax.axis_index("s")
        wid = sid * NC + cid
        base0 = wid * per
        row0 = sid * rows_per_tile
        idx = [idx0, idx1]
        rows = [rows0, rows1]
        lsem = [ls0, ls1]
        ssem = [ss0, ss1]

        def zbody(i, _):
            r = i // (D // 16)
            c = (i % (D // 16)) * 16
            stage[r, pl.ds(c, 16)] = jnp.zeros((16,), jnp.float32)
            return 0

        lax.fori_loop(0, SR * (D // 16), zbody, 0)
        for k in range(nsc):
            pltpu.sync_copy(stage, agg_sh.at[pl.ds(row0 + k * SR, SR)])
        plsc.subcore_barrier()

        def load(c, m):
            base = base0 + c * CH
            pltpu.async_copy(dst_hbm.at[pl.ds(e0 + base, CH)], idx[m], lsem[m])
            pltpu.async_copy(ne_hbm.at[pl.ds(base, CH)], rows[m], lsem[m])

        def wait_load(c, m):
            base = base0 + c * CH
            pltpu.make_async_copy(dst_hbm.at[pl.ds(e0 + base, CH)], idx[m], lsem[m]).wait()
            pltpu.make_async_copy(ne_hbm.at[pl.ds(base, CH)], rows[m], lsem[m]).wait()

        def scat(m):
            pltpu.async_copy(rows[m], agg_sh.at[idx[m]], ssem[m], add=True)

        def wait_scat(m):
            pltpu.make_async_copy(rows[m], agg_sh.at[idx[m]], ssem[m]).wait()

        npair = nch // 2
        load(0, 0)
        load(1, 1)

        def body(j, _):
            c0 = 2 * j
            for m in range(2):
                wait_load(c0 + m, m)
                scat(m)
            for m in range(2):
                wait_scat(m)
                extra = nch % 2 if m == 0 else 0

                @pl.when(j < npair - 1 + extra)
                def _():
                    load(c0 + 2 + m, m)

            return 0

        lax.fori_loop(0, npair, body, 0)

        if nch % 2:
            wait_load(nch - 1, 0)
            scat(0)
            wait_scat(0)

        if tail:
            base = base0 + nch * CH
            tr = rows0.at[pl.ds(0, tail)]
            pltpu.sync_copy(dst_hbm.at[pl.ds(e0 + base, tail)], tidx)
            pltpu.sync_copy(ne_hbm.at[pl.ds(base, tail)], tr)
            pltpu.sync_copy(tr, agg_sh.at[tidx], add=True)
        plsc.subcore_barrier()

        for k in range(nsc):
            pltpu.sync_copy(agg_sh.at[pl.ds(row0 + k * SR, SR)], stage)
            pltpu.sync_copy(stage, out_hbm.at[cid, sid, k])

    return scatter_k


def kernel(node_features, edge_features, edge_index, We1, be1, We2, be2,
           ge, bne, Wn1, bn1, Wn2, bn2, gn, bnn):
    N, D = node_features.shape
    E, DE = edge_features.shape
    src = edge_index[0]
    dst = edge_index[1]

    Wa, Wb, Wc = We1[:D], We1[D:2 * D], We1[2 * D:]
    Wn1x, Wn1g = Wn1[:D], Wn1[D:]
    be1r, be2r = be1.reshape(1, DE), be2.reshape(1, DE)
    ger, bner = ge.reshape(1, DE), bne.reshape(1, DE)
    bn1r, bn2r = bn1.reshape(1, D), bn2.reshape(1, D)
    gnr, bnnr = gn.reshape(1, D), bnn.reshape(1, D)

    BN = 1000
    full = lambda s: pl.BlockSpec(s, lambda i: tuple(0 for _ in s))

    pa, pb = pl.pallas_call(
        _proj_body,
        grid=(N // BN,),
        in_specs=[
            pl.BlockSpec((BN, D), lambda i: (i, 0)),
            full((D, DE)), full((D, DE)),
        ],
        out_specs=[pl.BlockSpec((BN, DE), lambda i: (i, 0)),
                   pl.BlockSpec((BN, DE), lambda i: (i, 0))],
        out_shape=[jax.ShapeDtypeStruct((N, DE), jnp.float32),
                   jax.ShapeDtypeStruct((N, DE), jnp.float32)],
    )(node_features, Wa, Wb)

    E2 = E // 2
    BE = 3200
    nb = E2 // BE

    def edge_call(phase, g, eo_prev):
        off = phase * nb
        specs = [
            pl.BlockSpec((BE, DE), lambda i: (i, 0)),
            pl.BlockSpec((BE, DE), lambda i, o=off: (i + o, 0)),
            full((DE, DE)), full((1, DE)), full((DE, DE)), full((1, DE)),
            full((1, DE)), full((1, DE)),
        ]
        args = [g, edge_features, Wc, be1r, We2, be2r, ger, bner]
        aliases = {}
        if eo_prev is not None:
            specs.append(pl.BlockSpec(memory_space=pl.ANY))
            args.append(eo_prev)
            aliases = {8: 1}
        return pl.pallas_call(
            _edge_body,
            grid=(nb,),
            in_specs=specs,
            out_specs=[pl.BlockSpec((BE, DE), lambda i: (i, 0)),
                       pl.BlockSpec((BE, DE), lambda i, o=off: (i + o, 0))],
            out_shape=[jax.ShapeDtypeStruct((E2, DE), jnp.float32),
                       jax.ShapeDtypeStruct((E, DE), jnp.float32)],
            input_output_aliases=aliases,
        )(*args)

    g_a = _make_gather(E2, DE, 0)(pa, pb, src, dst)
    g_b = _make_gather(E2, DE, E2)(pa, pb, src, dst)
    ne_a, eo_a = edge_call(0, g_a, None)
    parts_a = _make_scatter(E2, N, DE, 0)(ne_a, dst).reshape(NC, N, DE)
    ne_b, edge_out = edge_call(1, g_b, eo_a)
    parts_b = _make_scatter(E2, N, DE, E2)(ne_b, dst).reshape(NC, N, DE)

    node_out = pl.pallas_call(
        _node_body,
        grid=(N // BN,),
        in_specs=[
            pl.BlockSpec((BN, D), lambda i: (i, 0)),
            pl.BlockSpec((NC, BN, DE), lambda i: (0, i, 0)),
            pl.BlockSpec((NC, BN, DE), lambda i: (0, i, 0)),
            full((D, D)), full((DE, D)), full((1, D)),
            full((D, D)), full((1, D)), full((1, D)), full((1, D)),
        ],
        out_specs=pl.BlockSpec((BN, D), lambda i: (i, 0)),
        out_shape=jax.ShapeDtypeStruct((N, D), jnp.float32),
    )(node_features, parts_a, parts_b, Wn1x, Wn1g, bn1r, Wn2, bn2r, gnr, bnnr)

    return node_out, edge_out


# speedup vs baseline: 1.3249x; 1.0039x over previous
import functools

import jax
import jax.numpy as jnp
from jax import lax
from jax.experimental import pallas as pl
from jax.experimental.pallas import tpu as pltpu
from jax.experimental.pallas import tpu_sc as plsc

NC, NS = 2, 16
NW = NC * NS
CH = 128


def _proj_body(nf_ref, wa_ref, wb_ref, pa_ref, pb_ref):
    nf = nf_ref[...]
    pa_ref[...] = jnp.dot(nf, wa_ref[...], preferred_element_type=jnp.float32)
    pb_ref[...] = jnp.dot(nf, wb_ref[...], preferred_element_type=jnp.float32)


def _ln_rows(h, g, b, eps=1e-5):
    mu = jnp.mean(h, axis=-1, keepdims=True)
    var = jnp.mean((h - mu) ** 2, axis=-1, keepdims=True)
    return (h - mu) / jnp.sqrt(var + eps) * g + b


def _edge_body(g_ref, ef_ref, wc_ref, be1_ref, we2_ref, be2_ref,
               ge_ref, bne_ref, *refs):
    ne_ref, eo_ref = refs[-2], refs[-1]
    ef = ef_ref[...]
    h = (g_ref[...] + be1_ref[...]
         + jnp.dot(ef, wc_ref[...], preferred_element_type=jnp.float32))
    h = jnp.maximum(h, 0.0)
    h2 = jnp.dot(h, we2_ref[...], preferred_element_type=jnp.float32) + be2_ref[...]
    ne = _ln_rows(h2, ge_ref[...], bne_ref[...])
    ne_ref[...] = ne
    eo_ref[...] = ne + ef


def _node_body(nf_ref, agga_ref, aggb_ref, wx_ref, wg_ref, bn1_ref, wn2_ref,
               bn2_ref, gn_ref, bnn_ref, out_ref):
    nf = nf_ref[...]
    agg = agga_ref[0] + agga_ref[1] + aggb_ref[0] + aggb_ref[1]
    h = (jnp.dot(nf, wx_ref[...], preferred_element_type=jnp.float32)
         + jnp.dot(agg, wg_ref[...], preferred_element_type=jnp.float32)
         + bn1_ref[...])
    h = jnp.maximum(h, 0.0)
    h2 = jnp.dot(h, wn2_ref[...], preferred_element_type=jnp.float32) + bn2_ref[...]
    out_ref[...] = _ln_rows(h2, gn_ref[...], bnn_ref[...]) + nf


NSLOT = 3


def _make_gather(E, D, e0=0, Eout=None):
    per = E // NW
    nch = per // CH
    tail = per - nch * CH
    ntrip = nch // NSLOT
    Eout = E if Eout is None else Eout
    mesh = plsc.VectorSubcoreMesh(core_axis_name="c", subcore_axis_name="s",
                                  num_cores=NC, num_subcores=NS)

    scratch = []
    for _ in range(NSLOT):
        scratch += [pltpu.VMEM((CH,), jnp.int32),
                    pltpu.VMEM((CH, D), jnp.float32),
                    pltpu.VMEM((CH,), jnp.int32),
                    pltpu.VMEM((CH, D), jnp.float32),
                    pltpu.SemaphoreType.DMA,
                    pltpu.SemaphoreType.DMA,
                    pltpu.SemaphoreType.DMA]

    @functools.partial(
        pl.kernel,
        out_type=jax.ShapeDtypeStruct((Eout, D), jnp.float32),
        mesh=mesh,
        scratch_types=scratch,
    )
    def gather_k(pa_hbm, pb_hbm, src_hbm, dst_hbm, g_hbm, *bufs):
        slot = [bufs[7 * m:7 * (m + 1)] for m in range(NSLOT)]
        wid = lax.axis_index("s") * NC + lax.axis_index("c")
        base0 = wid * per
        gsrc0 = e0 + base0

        def issue(c, m):
            idx1, rows1, idx2, rows2, gs1, gs2, _ = slot[m]
            base = gsrc0 + c * CH
            pltpu.sync_copy(src_hbm.at[pl.ds(base, CH)], idx1)
            pltpu.sync_copy(dst_hbm.at[pl.ds(base, CH)], idx2)
            pltpu.async_copy(pa_hbm.at[idx1], rows1, gs1)
            pltpu.async_copy(pb_hbm.at[idx2], rows2, gs2)

        def wait_gather(m):
            idx1, rows1, idx2, rows2, gs1, gs2, _ = slot[m]
            pltpu.make_async_copy(pa_hbm.at[idx1], rows1, gs1).wait()
            pltpu.make_async_copy(pb_hbm.at[idx2], rows2, gs2).wait()

        def fuse_add(m):
            _, rows1, _, rows2, _, _, _ = slot[m]

            def rbody(r, _):
                for c in range(D // 16):
                    sl = pl.ds(16 * c, 16)
                    plsc.addupdate(rows1.at[r, sl], rows2[r, sl])
                return 0

            lax.fori_loop(0, CH, rbody, 0)

        def writeback(c, m):
            _, rows1, _, _, _, _, ws = slot[m]
            base = base0 + c * CH
            pltpu.async_copy(rows1, g_hbm.at[pl.ds(base, CH)], ws)

        def wait_writeback(c, m):
            _, rows1, _, _, _, _, ws = slot[m]
            base = base0 + c * CH
            pltpu.make_async_copy(rows1, g_hbm.at[pl.ds(base, CH)], ws).wait()

        for m in range(NSLOT):
            issue(m, m)

        def body(j, _):
            c0 = j * NSLOT
            for m in range(NSLOT):
                wait_gather(m)
                fuse_add(m)
                writeback(c0 + m, m)

            @pl.when(j < ntrip - 1)
            def _():
                for m in range(NSLOT):
                    wait_writeback(c0 + m, m)
                    issue(c0 + NSLOT + m, m)

            return 0

        lax.fori_loop(0, ntrip, body, 0)
        for m in range(NSLOT):
            wait_writeback((ntrip - 1) * NSLOT + m, m)

        idx1, rows1, idx2, rows2, gs1, gs2, _ = slot[0]
        base = base0 + nch * CH
        gsbase = gsrc0 + nch * CH
        ti1, ti2 = idx1.at[pl.ds(0, tail)], idx2.at[pl.ds(0, tail)]
        tr1, tr2 = rows1.at[pl.ds(0, tail)], rows2.at[pl.ds(0, tail)]
        pltpu.sync_copy(src_hbm.at[pl.ds(gsbase, tail)], ti1)
        pltpu.sync_copy(dst_hbm.at[pl.ds(gsbase, tail)], ti2)
        pltpu.async_copy(pa_hbm.at[ti1], tr1, gs1).wait()
        pltpu.async_copy(pb_hbm.at[ti2], tr2, gs2).wait()

        def tbody(r, _):
            for c in range(D // 16):
                sl = pl.ds(16 * c, 16)
                plsc.addupdate(rows1.at[r, sl], rows2[r, sl])
            return 0

        lax.fori_loop(0, tail, tbody, 0)
        pltpu.sync_copy(tr1, g_hbm.at[pl.ds(base, tail)])

    return gather_k


def _make_scatter(E, N, D, e0=0):
    per = E // NW
    nch = per // CH
    tail = per - nch * CH
    rows_per_tile = N // NS
    SR = 125
    nsc = rows_per_tile // SR
    mesh = plsc.VectorSubcoreMesh(core_axis_name="c", subcore_axis_name="s",
                                  num_cores=NC, num_subcores=NS)

    @functools.partial(
        pl.kernel,
        out_type=jax.ShapeDtypeStruct((NC, NS, nsc, SR, D), jnp.float32),
        mesh=mesh,
        scratch_types=[
            pltpu.VMEM((CH,), jnp.int32),
            pltpu.VMEM((CH, D), jnp.float32),
            pltpu.VMEM((CH,), jnp.int32),
            pltpu.VMEM((CH, D), jnp.float32),
            pltpu.VMEM((SR, D), jnp.float32),
            pltpu.VMEM_SHARED((N, D), jnp.float32),
            pltpu.VMEM((tail or 8,), jnp.int32),
            pltpu.SemaphoreType.DMA,
            pltpu.SemaphoreType.DMA,
            pltpu.SemaphoreType.DMA,
            pltpu.SemaphoreType.DMA,
            pltpu.SemaphoreType.DMA,
        ],
    )
    def scatter_k(ne_hbm, dst_hbm, out_hbm, idx0, rows0, idx1, rows1,
                  stage, agg_sh, tidx, ls0, ls1, ss0, ss1, zs):
        cid = lax.axis_index("c")
        sid = lax.axis_index("s")
        wid = sid * NC + cid
        base0 = wid * per
        row0 = sid * rows_per_tile
        idx = [idx0, idx1]
        rows = [rows0, rows1]
        lsem = [ls0, ls1]
        ssem = [ss0, ss1]

        def load(c, m):
            base = base0 + c * CH
            pltpu.async_copy(dst_hbm.at[pl.ds(e0 + base, CH)], idx[m], lsem[m])
            pltpu.async_copy(ne_hbm.at[pl.ds(base, CH)], rows[m], lsem[m])

        def wait_load(c, m):
            base = base0 + c * CH
            pltpu.make_async_copy(dst_hbm.at[pl.ds(e0 + base, CH)], idx[m], lsem[m]).wait()
            pltpu.make_async_copy(ne_hbm.at[pl.ds(base, CH)], rows[m], lsem[m]).wait()

        def scat(m):
            pltpu.async_copy(rows[m], agg_sh.at[idx[m]], ssem[m], add=True)

        def wait_scat(m):
            pltpu.make_async_copy(rows[m], agg_sh.at[idx[m]], ssem[m]).wait()

        npair = nch // 2
        load(0, 0)
        load(1, 1)

        def zbody(i, _):
            r = i // (D // 16)
            c = (i % (D // 16)) * 16
            stage[r, pl.ds(c, 16)] = jnp.zeros((16,), jnp.float32)
            return 0

        lax.fori_loop(0, SR * (D // 16), zbody, 0)
        for k in range(nsc):
            pltpu.async_copy(stage, agg_sh.at[pl.ds(row0 + k * SR, SR)], zs)
        for k in range(nsc):
            pltpu.make_async_copy(stage, agg_sh.at[pl.ds(row0 + k * SR, SR)],
                                  zs).wait()
        plsc.subcore_barrier()

        def body(j, _):
            c0 = 2 * j
            for m in range(2):
                wait_load(c0 + m, m)
                scat(m)
            for m in range(2):
                wait_scat(m)
                extra = nch % 2 if m == 0 else 0

                @pl.when(j < npair - 1 + extra)
                def _():
                    load(c0 + 2 + m, m)

            return 0

        lax.fori_loop(0, npair, body, 0)

        if nch % 2:
            wait_load(nch - 1, 0)
            scat(0)
            wait_scat(0)

        if tail:
            base = base0 + nch * CH
            tr = rows0.at[pl.ds(0, tail)]
            pltpu.sync_copy(dst_hbm.at[pl.ds(e0 + base, tail)], tidx)
            pltpu.sync_copy(ne_hbm.at[pl.ds(base, tail)], tr)
            pltpu.sync_copy(tr, agg_sh.at[tidx], add=True)
        plsc.subcore_barrier()

        for k in range(nsc):
            pltpu.sync_copy(agg_sh.at[pl.ds(row0 + k * SR, SR)], stage)
            pltpu.sync_copy(stage, out_hbm.at[cid, sid, k])

    return scatter_k


def kernel(node_features, edge_features, edge_index, We1, be1, We2, be2,
           ge, bne, Wn1, bn1, Wn2, bn2, gn, bnn):
    N, D = node_features.shape
    E, DE = edge_features.shape
    src = edge_index[0]
    dst = edge_index[1]

    Wa, Wb, Wc = We1[:D], We1[D:2 * D], We1[2 * D:]
    Wn1x, Wn1g = Wn1[:D], Wn1[D:]
    be1r, be2r = be1.reshape(1, DE), be2.reshape(1, DE)
    ger, bner = ge.reshape(1, DE), bne.reshape(1, DE)
    bn1r, bn2r = bn1.reshape(1, D), bn2.reshape(1, D)
    gnr, bnnr = gn.reshape(1, D), bnn.reshape(1, D)

    BN = 1000
    full = lambda s: pl.BlockSpec(s, lambda i: tuple(0 for _ in s))

    pa, pb = pl.pallas_call(
        _proj_body,
        grid=(N // BN,),
        in_specs=[
            pl.BlockSpec((BN, D), lambda i: (i, 0)),
            full((D, DE)), full((D, DE)),
        ],
        out_specs=[pl.BlockSpec((BN, DE), lambda i: (i, 0)),
                   pl.BlockSpec((BN, DE), lambda i: (i, 0))],
        out_shape=[jax.ShapeDtypeStruct((N, DE), jnp.float32),
                   jax.ShapeDtypeStruct((N, DE), jnp.float32)],
    )(node_features, Wa, Wb)

    E2 = E // 2
    BE = 3200
    nb = E2 // BE

    def edge_call(phase, g, eo_prev):
        off = phase * nb
        specs = [
            pl.BlockSpec((BE, DE), lambda i: (i, 0)),
            pl.BlockSpec((BE, DE), lambda i, o=off: (i + o, 0)),
            full((DE, DE)), full((1, DE)), full((DE, DE)), full((1, DE)),
            full((1, DE)), full((1, DE)),
        ]
        args = [g, edge_features, Wc, be1r, We2, be2r, ger, bner]
        aliases = {}
        if eo_prev is not None:
            specs.append(pl.BlockSpec(memory_space=pl.ANY))
            args.append(eo_prev)
            aliases = {8: 1}
        return pl.pallas_call(
            _edge_body,
            grid=(nb,),
            in_specs=specs,
            out_specs=[pl.BlockSpec((BE, DE), lambda i: (i, 0)),
                       pl.BlockSpec((BE, DE), lambda i, o=off: (i + o, 0))],
            out_shape=[jax.ShapeDtypeStruct((E2, DE), jnp.float32),
                       jax.ShapeDtypeStruct((E, DE), jnp.float32)],
            input_output_aliases=aliases,
        )(*args)

    g_a = _make_gather(E2, DE, 0)(pa, pb, src, dst)
    g_b = _make_gather(E2, DE, E2)(pa, pb, src, dst)
    ne_a, eo_a = edge_call(0, g_a, None)
    parts_a = _make_scatter(E2, N, DE, 0)(ne_a, dst).reshape(NC, N, DE)
    ne_b, edge_out = edge_call(1, g_b, eo_a)
    parts_b = _make_scatter(E2, N, DE, E2)(ne_b, dst).reshape(NC, N, DE)

    node_out = pl.pallas_call(
        _node_body,
        grid=(N // BN,),
        in_specs=[
            pl.BlockSpec((BN, D), lambda i: (i, 0)),
            pl.BlockSpec((NC, BN, DE), lambda i: (0, i, 0)),
            pl.BlockSpec((NC, BN, DE), lambda i: (0, i, 0)),
            full((D, D)), full((DE, D)), full((1, D)),
            full((D, D)), full((1, D)), full((1, D)), full((1, D)),
        ],
        out_specs=pl.BlockSpec((BN, D), lambda i: (i, 0)),
        out_shape=jax.ShapeDtypeStruct((N, D), jnp.float32),
    )(node_features, parts_a, parts_b, Wn1x, Wn1g, bn1r, Wn2, bn2r, gnr, bnnr)

    return node_out, edge_out
